# baseline (speedup 1.0000x reference)
import functools

import jax
import jax.numpy as jnp
from jax import lax
from jax.experimental import pallas as pl
from jax.experimental.pallas import tpu as pltpu
from jax.experimental.pallas import tpu_sc as plsc

_NUM_TABLES = 26
_VOCAB = 100000
_EMB = 32
_BATCH = 16384
_HIST = 20

_NC = 2
_NS = 16
_T_PER_C = _NUM_TABLES // _NC
_B_PER_W = _BATCH // _NS
_CHUNK = 64
_N_CHUNK = _B_PER_W // _CHUNK
_IDX_PER_CHUNK = _CHUNK * _HIST

_STRIPE = 400
_S_PER_T = _VOCAB // _STRIPE
_S_PER_C = _T_PER_C * _S_PER_T
_K_MAX = (_S_PER_C + _NS - 1) // _NS


def _sc_embedding_bag(idx_t, tab_t):
    mesh = plsc.VectorSubcoreMesh(core_axis_name="c", subcore_axis_name="s")

    @functools.partial(
        pl.kernel,
        mesh=mesh,
        compiler_params=pltpu.CompilerParams(use_tc_tiling_on_sc=False, needs_layout_passes=False),
        out_type=(
            jax.ShapeDtypeStruct((_BATCH, _NUM_TABLES * _EMB), jnp.float32),
            jax.ShapeDtypeStruct((_NUM_TABLES, _VOCAB, _EMB), jnp.float32),
        ),
        scratch_types=[
            pltpu.VMEM((_EMB, _STRIPE), jnp.float32),
            pltpu.VMEM((_STRIPE, _EMB), jnp.float32),
            pltpu.VMEM((_IDX_PER_CHUNK,), jnp.int32),
            pltpu.VMEM((_IDX_PER_CHUNK, _EMB), jnp.float32),
            pltpu.VMEM((_CHUNK, _EMB), jnp.float32),
            pltpu.SemaphoreType.DMA,
            pltpu.SemaphoreType.DMA,
        ],
    )
    def k(idx_hbm, tab_hbm, out_hbm, scr_hbm, plane_v, rows_v, idx_v, gat_v,
          acc_v, sem_i, sem_g):
        cid = lax.axis_index("c")
        sid = lax.axis_index("s")
        t_base = cid * _T_PER_C

        _NB = _STRIPE // 16

        def stripe_body(kk, _):
            s = kk * _NS + sid
            t = t_base + s // _S_PER_T
            v0 = (s % _S_PER_T) * _STRIPE
            pltpu.sync_copy(tab_hbm.at[t, :, pl.ds(v0, _STRIPE)],
                            plane_v)
            for i in range(_NB):
                row_ids = lax.iota(jnp.int32, 16) + i * 16
                for e in range(_EMB):
                    x = plane_v[e, pl.ds(i * 16, 16)]
                    plsc.store_scatter(
                        rows_v, [row_ids, jnp.full((16,), e, jnp.int32)], x)
            pltpu.sync_copy(rows_v,
                            scr_hbm.at[t, pl.ds(v0, _STRIPE), :])
            return 0

        n_k = jnp.where(sid < _S_PER_C % _NS, _K_MAX, _K_MAX - 1)
        lax.fori_loop(0, n_k, stripe_body, 0)
        plsc.subcore_barrier()

        def unit_body(u, _):
            t = t_base + u // _N_CHUNK
            bstart = sid * _B_PER_W + (u % _N_CHUNK) * _CHUNK

            copies = [
                pltpu.async_copy(
                    idx_hbm.at[t, h, pl.ds(bstart, _CHUNK)],
                    idx_v.at[pl.ds(h * _CHUNK, _CHUNK)], sem_i)
                for h in range(_HIST)
            ]
            for cp in copies:
                cp.wait()

            pltpu.async_copy(scr_hbm.at[t].at[idx_v], gat_v, sem_g).wait()

            def bag(j, _):
                lo = gat_v[j, 0:16]
                hi = gat_v[j, 16:32]
                for h in range(1, _HIST):
                    lo = lo + gat_v[h * _CHUNK + j, 0:16]
                    hi = hi + gat_v[h * _CHUNK + j, 16:32]
                acc_v[j, 0:16] = lo
                acc_v[j, 16:32] = hi
                return 0

            lax.fori_loop(0, _CHUNK, bag, 0)

            pltpu.sync_copy(
                acc_v,
                out_hbm.at[pl.ds(bstart, _CHUNK), pl.ds(t * _EMB, _EMB)])
            return 0

        lax.fori_loop(0, _T_PER_C * _N_CHUNK, unit_body, 0)

    return k(idx_t, tab_t)


def kernel(indices, tables):
    idx_t = jnp.transpose(indices, (0, 2, 1))
    tab_t = jnp.transpose(tables, (0, 2, 1))
    out, _ = _sc_embedding_bag(idx_t, tab_t)
    return out


# speedup vs baseline: 1.9703x; 1.9703x over previous
import functools

import jax
import jax.numpy as jnp
from jax import lax
from jax.experimental import pallas as pl
from jax.experimental.pallas import tpu as pltpu
from jax.experimental.pallas import tpu_sc as plsc

_NUM_TABLES = 26
_VOCAB = 100000
_EMB = 32
_BATCH = 16384
_HIST = 20

_NC = 2
_NS = 16
_T_PER_C = _NUM_TABLES // _NC
_B_PER_W = _BATCH // _NS
_CHUNK = 64
_N_CHUNK = _B_PER_W // _CHUNK
_ROWS = _CHUNK * _HIST
_N_UNIT = _T_PER_C * _N_CHUNK


def _sc_embedding_bag(idx_t, tables):
    mesh = plsc.VectorSubcoreMesh(core_axis_name="c", subcore_axis_name="s")

    @functools.partial(
        pl.kernel,
        mesh=mesh,
        compiler_params=pltpu.CompilerParams(
            use_tc_tiling_on_sc=False, needs_layout_passes=False),
        out_type=jax.ShapeDtypeStruct((_BATCH, _NUM_TABLES * _EMB),
                                      jnp.float32),
        scratch_types=[
            pltpu.VMEM((_ROWS,), jnp.int32),
            pltpu.VMEM((_ROWS,), jnp.int32),
            pltpu.VMEM((_ROWS, _EMB), jnp.float32),
            pltpu.VMEM((_ROWS, _EMB), jnp.float32),
            pltpu.VMEM((_CHUNK, _EMB), jnp.float32),
            pltpu.VMEM((_CHUNK, _EMB), jnp.float32),
            pltpu.SemaphoreType.DMA,
            pltpu.SemaphoreType.DMA,
            pltpu.SemaphoreType.DMA,
            pltpu.SemaphoreType.DMA,
            pltpu.SemaphoreType.DMA,
            pltpu.SemaphoreType.DMA,
        ],
    )
    def k(idx_hbm, tab_hbm, out_hbm, idx0, idx1, gat0, gat1, acc0, acc1,
          si0, si1, sg0, sg1, so0, so1):
        cid = lax.axis_index("c")
        sid = lax.axis_index("s")
        t_base = cid * _T_PER_C

        def unit_tb(u):
            return t_base + u // _N_CHUNK, sid * _B_PER_W + (
                u % _N_CHUNK) * _CHUNK

        def issue_idx(u, idxb, sem):
            t, bstart = unit_tb(u)
            for h in range(_HIST):
                pltpu.async_copy(
                    idx_hbm.at[t, h, pl.ds(bstart, _CHUNK)],
                    idxb.at[pl.ds(h * _CHUNK, _CHUNK)], sem)

        def wait_idx(idxb, sem):
            pltpu.make_async_copy(
                idx_hbm.at[0, 0, pl.ds(0, _ROWS)], idxb, sem).wait()

        def issue_gather(u, idxb, gatb, sem):
            t, _ = unit_tb(u)
            pltpu.async_copy(tab_hbm.at[t].at[idxb], gatb, sem)

        def wait_gather(gatb, sem):
            pltpu.make_async_copy(
                tab_hbm.at[0, pl.ds(0, _ROWS), :], gatb, sem).wait()

        def reduce_and_out(u, gatb, accb, sem):
            def bag(j, _):
                lo = gatb[j, 0:16]
                hi = gatb[j, 16:32]
                for h in range(1, _HIST):
                    lo = lo + gatb[h * _CHUNK + j, 0:16]
                    hi = hi + gatb[h * _CHUNK + j, 16:32]
                accb[j, 0:16] = lo
                accb[j, 16:32] = hi
                return 0

            lax.fori_loop(0, _CHUNK, bag, 0)
            t, bstart = unit_tb(u)
            pltpu.async_copy(
                accb,
                out_hbm.at[pl.ds(bstart, _CHUNK), pl.ds(t * _EMB, _EMB)],
                sem)

        def wait_out(accb, sem):
            pltpu.make_async_copy(
                out_hbm.at[pl.ds(0, _CHUNK), pl.ds(0, _EMB)], accb,
                sem).wait()

        pltpu.async_copy(out_hbm.at[pl.ds(0, _CHUNK), pl.ds(0, _EMB)],
                         acc0, so0)
        pltpu.async_copy(out_hbm.at[pl.ds(0, _CHUNK), pl.ds(0, _EMB)],
                         acc1, so1)
        issue_idx(0, idx0, si0)
        issue_idx(1, idx1, si1)
        wait_idx(idx0, si0)
        issue_gather(0, idx0, gat0, sg0)

        def body(kk, _):
            u = 2 * kk
            wait_gather(gat0, sg0)
            issue_idx(u + 2, idx0, si0)
            wait_idx(idx1, si1)
            issue_gather(u + 1, idx1, gat1, sg1)
            wait_out(acc0, so0)
            reduce_and_out(u, gat0, acc0, so0)
            wait_gather(gat1, sg1)
            issue_idx(u + 3, idx1, si1)
            wait_idx(idx0, si0)
            issue_gather(u + 2, idx0, gat0, sg0)
            wait_out(acc1, so1)
            reduce_and_out(u + 1, gat1, acc1, so1)
            return 0

        lax.fori_loop(0, (_N_UNIT - 2) // 2, body, 0)

        u = _N_UNIT - 2
        wait_gather(gat0, sg0)
        wait_idx(idx1, si1)
        issue_gather(u + 1, idx1, gat1, sg1)
        wait_out(acc0, so0)
        reduce_and_out(u, gat0, acc0, so0)
        wait_gather(gat1, sg1)
        wait_out(acc1, so1)
        reduce_and_out(u + 1, gat1, acc1, so1)
        wait_out(acc0, so0)
        wait_out(acc1, so1)

    return k(idx_t, tables)


def kernel(indices, tables):
    idx_t = jnp.transpose(indices, (0, 2, 1))
    return _sc_embedding_bag(idx_t, tables)
